# baseline (speedup 1.0000x reference)
import functools

import jax
import jax.numpy as jnp
from jax import lax
from jax.experimental import pallas as pl
from jax.experimental.pallas import tpu as pltpu
from jax.experimental.pallas import tpu_sc as plsc

B = 16384
D = 64
NC = 2
NS = 16
NW = NC * NS
BPW = B // NW
CHUNK = 128
NCHUNK = BPW // CHUNK


def _body(user_r, movie_r, uids, mids, out_hbm,
          uraw_v, mraw_v, uq_v, mq_v, ubuf, mbuf, out_v, sem0, sem1):
    w = lax.axis_index("c") * NS + lax.axis_index("s")

    pltpu.sync_copy(uids.at[pl.ds(w * NCHUNK, NCHUNK)], uraw_v)
    pltpu.sync_copy(mids.at[pl.ds(w * NCHUNK, NCHUNK)], mraw_v)

    lane = lax.iota(jnp.int32, 16)
    sems = (sem0, sem1)

    def halve(i, carry):
        c = i // 8
        s = (i % 8) * 16
        uq_v[c, pl.ds(s, 16)] = uraw_v[c, pl.ds(s, 16)] >> 1
        mq_v[c, pl.ds(s, 16)] = mraw_v[c, pl.ds(s, 16)] >> 1
        return carry

    lax.fori_loop(0, NCHUNK * 8, halve, 0)

    def issue(c, par):
        pltpu.async_copy(user_r.at[uq_v.at[c]], ubuf.at[par], sems[par])
        pltpu.async_copy(movie_r.at[mq_v.at[c]], mbuf.at[par], sems[par])

    def drain(par):
        pltpu.make_async_copy(user_r.at[pl.ds(0, CHUNK)], ubuf.at[par],
                              sems[par]).wait()
        pltpu.make_async_copy(movie_r.at[pl.ds(0, CHUNK)], mbuf.at[par],
                              sems[par]).wait()

    def compute(c, par):
        for g in range(CHUNK // 16):
            uvec = uraw_v[c, pl.ds(g * 16, 16)]
            mvec = mraw_v[c, pl.ds(g * 16, 16)]
            uoff = (uvec & 1) * 64
            moff = (mvec & 1) * 64
            res = jnp.zeros((16,), jnp.float32)
            for e in range(16):
                j = g * 16 + e
                uo = jnp.sum(jnp.where(lane == e, uoff, 0))
                mo = jnp.sum(jnp.where(lane == e, moff, 0))
                acc = jnp.zeros((16,), jnp.float32)
                for k in range(D // 16):
                    u = ubuf[par, j, pl.ds(uo + k * 16, 16)]
                    m = mbuf[par, j, pl.ds(mo + k * 16, 16)]
                    acc = acc + u * m
                res = jnp.where(lane == e, jnp.sum(acc), res)
            out_v[pl.ds(c * CHUNK + g * 16, 16)] = res

    def step(i, carry):
        for par in (0, 1):
            c = i * 2 + par

            @pl.when(c < NCHUNK)
            def _():
                issue(c, par)

            @pl.when(jnp.logical_and(c > 0, c <= NCHUNK))
            def _():
                drain(1 - par)
                compute(c - 1, 1 - par)

        return carry

    lax.fori_loop(0, (NCHUNK + 2) // 2, step, 0)

    pltpu.sync_copy(out_v, out_hbm.at[pl.ds(w * BPW, BPW)])


@jax.jit
def _mf_kernel(user_ids, movie_ids, user_emb, movie_emb):
    nu = user_emb.shape[0]
    nm = movie_emb.shape[0]
    user_r = user_emb.reshape(nu // 2, 2 * D)
    movie_r = movie_emb.reshape(nm // 2, 2 * D)
    uids = user_ids.astype(jnp.int32).reshape(NW * NCHUNK, CHUNK)
    mids = movie_ids.astype(jnp.int32).reshape(NW * NCHUNK, CHUNK)

    mesh = plsc.VectorSubcoreMesh(core_axis_name="c", subcore_axis_name="s")
    run = functools.partial(
        pl.kernel,
        mesh=mesh,
        compiler_params=pltpu.CompilerParams(needs_layout_passes=False),
        out_type=jax.ShapeDtypeStruct((B,), jnp.float32),
        scratch_types=[
            pltpu.VMEM((NCHUNK, CHUNK), jnp.int32),
            pltpu.VMEM((NCHUNK, CHUNK), jnp.int32),
            pltpu.VMEM((NCHUNK, CHUNK), jnp.int32),
            pltpu.VMEM((NCHUNK, CHUNK), jnp.int32),
            pltpu.VMEM((2, CHUNK, 2 * D), jnp.float32),
            pltpu.VMEM((2, CHUNK, 2 * D), jnp.float32),
            pltpu.VMEM((BPW,), jnp.float32),
            pltpu.SemaphoreType.DMA,
            pltpu.SemaphoreType.DMA,
        ],
    )(_body)
    return run(user_r, movie_r, uids, mids)


def kernel(user_ids, movie_ids, user_emb, movie_emb):
    return _mf_kernel(user_ids, movie_ids, user_emb, movie_emb)


# speedup vs baseline: 1.5105x; 1.5105x over previous
import functools

import jax
import jax.numpy as jnp
from jax import lax
from jax.experimental import pallas as pl
from jax.experimental.pallas import tpu as pltpu
from jax.experimental.pallas import tpu_sc as plsc

B = 16384
D = 64
NU = 1000000
NM = 100000
NC = 2
NS = 16
NW = NC * NS
BPW = B // NW

NBU = (NU + 127) // 128
NBM = (NM + 127) // 128
UTAIL = NU - (NBU - 1) * 128
MTAIL = NM - (NBM - 1) * 128
UBASE, UEXTRA = NBU // NW, NBU % NW
MBASE, MEXTRA = NBM // NW, NBM % NW
RING = 4
MCAP = 4096
DUMP = B
OROWS = B + 16


def _gather_body(ut, mt, uids_h, mids_h, uvecs, mvecs,
                 ids_all, match_ids, match_pos, chunk, tailu_v, tailm_v,
                 stage_v, sidx_v, out_v_unused,
                 semr0, semr1, semr2, semr3, sem_s):
    del out_v_unused
    w = lax.axis_index("c") * NS + lax.axis_index("s")
    lane = lax.iota(jnp.int32, 16)
    sems = (semr0, semr1, semr2, semr3)

    def sum16(vec):
        return jnp.sum(vec)

    def extract(vec, onehot):
        return jnp.sum(jnp.where(onehot, vec, 0))

    def phase(table, ids_src, out_ref, nb_all, base, extra, tail_w, tail_v):
        blk0 = w * base + jnp.minimum(w, extra)
        nblk = base + jnp.where(w < extra, 1, 0)
        owns_tail = blk0 + nblk >= nb_all
        nfull = nblk - jnp.where(owns_tail, 1, 0)

        pltpu.sync_copy(ids_src, ids_all)

        lo = blk0 * 128
        hi = (blk0 + nblk) * 128

        def mstep(i, offv):
            v = ids_all[pl.ds(i * 16, 16)]
            m = jnp.logical_and(v >= lo, v < hi)
            csum = plsc.cumsum(m.astype(jnp.int32))
            idx = offv + csum - 1
            m = jnp.logical_and(m, idx < MCAP)
            plsc.store_scatter(match_ids, [idx], v, mask=m)
            plsc.store_scatter(match_pos, [idx], i * 16 + lane, mask=m)
            return offv + plsc.all_reduce_population_count(m)

        offv = lax.fori_loop(0, B // 16, mstep, jnp.zeros((16,), jnp.int32))
        nm_s = sum16(jnp.where(lane == 0, offv, 0))
        nmv = offv

        def process(gat, c0, width, carry, active):
            sidx_vec, bcv = carry
            nv = jnp.where(active, (nm_s + 15) // 16, 0)

            def vstep(v, carry):
                sidx_vec, bcv = carry
                ids16 = match_ids[pl.ds(v * 16, 16)]
                pos16 = match_pos[pl.ds(v * 16, 16)]
                valid = (v * 16 + lane) < nmv
                inblk = jnp.logical_and(
                    valid,
                    jnp.logical_and(ids16 >= c0, ids16 < c0 + width))

                def wcond(st):
                    msk, _, _ = st
                    return jnp.sum(jnp.where(msk, 1, 0)) > 0

                def wbody(st):
                    msk, sidx_vec, bcv = st
                    f = plsc.all_reduce_ffs(msk)
                    oh = lane == f
                    id_s = extract(ids16, oh)
                    b_s = extract(pos16, oh)
                    lc = id_s - c0
                    bc_s = sum16(jnp.where(lane == 0, bcv, 0))
                    lcv = jnp.broadcast_to(lc, (16,))
                    for k4 in range(D // 16):
                        g = gat(k4 * 16 + lane, lcv)
                        stage_v[bc_s, pl.ds(k4 * 16, 16)] = g
                    sidx_vec = jnp.where(lane == bcv, b_s, sidx_vec)
                    bcv = bcv + 1

                    @pl.when(bc_s == 15)
                    def _():
                        sidx_v[pl.ds(0, 16)] = sidx_vec
                        pltpu.async_copy(stage_v, out_ref.at[sidx_v],
                                         sem_s).wait()

                    sidx_vec = jnp.where(bc_s == 15,
                                         jnp.full((16,), DUMP, jnp.int32),
                                         sidx_vec)
                    bcv = jnp.where(bc_s == 15, 0, bcv)
                    return (jnp.logical_and(msk, jnp.logical_not(oh)),
                            sidx_vec, bcv)

                _, sidx_vec, bcv = lax.while_loop(
                    wcond, wbody, (inblk, sidx_vec, bcv))
                return (sidx_vec, bcv)

            return lax.fori_loop(0, nv, vstep, (sidx_vec, bcv))

        def issue(slot, bl):
            c0 = pl.multiple_of((blk0 + bl) * 128, 128)
            pltpu.async_copy(table.at[:, pl.ds(c0, 128)], chunk.at[slot],
                             sems[slot])

        for slot in range(RING):
            @pl.when(slot < nfull)
            def _():
                issue(slot, slot)

        def rstep(i, carry):
            for slot in range(RING):
                bl = i * RING + slot

                @pl.when(bl < nfull)
                def _():
                    pltpu.make_async_copy(
                        table.at[:, pl.ds(0, 128)], chunk.at[slot],
                        sems[slot]).wait()

                c0 = (blk0 + bl) * 128
                slot_v = jnp.full((16,), slot, jnp.int32)
                carry = process(
                    lambda rows, cols, sv=slot_v: plsc.load_gather(
                        chunk, [sv, rows, cols]),
                    c0, 128, carry, bl < nfull)

                @pl.when(bl + RING < nfull)
                def _():
                    issue(slot, bl + RING)
            return carry

        carry = (jnp.full((16,), DUMP, jnp.int32), jnp.zeros((16,), jnp.int32))
        nsteps = (nfull + RING - 1) // RING
        carry = lax.fori_loop(0, nsteps, rstep, carry)

        tc0 = (nb_all - 1) * 128

        @pl.when(owns_tail)
        def _():
            pltpu.sync_copy(table.at[:, pl.ds(tc0, tail_w)], tail_v)

        carry = process(
            lambda rows, cols: plsc.load_gather(tail_v, [rows, cols]),
            (nb_all - 1) * 128, tail_w, carry, owns_tail)

        sidx_vec, bcv = carry
        bc_s = sum16(jnp.where(lane == 0, bcv, 0))

        @pl.when(bc_s > 0)
        def _():
            sidx_v[pl.ds(0, 16)] = sidx_vec
            pltpu.async_copy(stage_v, out_ref.at[sidx_v], sem_s).wait()

    phase(ut, uids_h, uvecs, NBU, UBASE, UEXTRA, UTAIL, tailu_v)
    phase(mt, mids_h, mvecs, NBM, MBASE, MEXTRA, MTAIL, tailm_v)


def _dot_body(uvecs, mvecs, out_hbm, ubuf, mbuf, out_v, sem0, sem1):
    w = lax.axis_index("c") * NS + lax.axis_index("s")
    base = w * BPW
    CH = 128
    NCH = BPW // CH
    lane = lax.iota(jnp.int32, 16)
    sems = (sem0, sem1)

    def issue(c, par):
        pltpu.async_copy(uvecs.at[pl.ds(base + c * CH, CH)], ubuf.at[par],
                         sems[par])
        pltpu.async_copy(mvecs.at[pl.ds(base + c * CH, CH)], mbuf.at[par],
                         sems[par])

    def drain(par):
        pltpu.make_async_copy(uvecs.at[pl.ds(0, CH)], ubuf.at[par],
                              sems[par]).wait()
        pltpu.make_async_copy(mvecs.at[pl.ds(0, CH)], mbuf.at[par],
                              sems[par]).wait()

    def compute(c, par):
        for g in range(CH // 16):
            res = jnp.zeros((16,), jnp.float32)
            for e in range(16):
                j = g * 16 + e
                acc = jnp.zeros((16,), jnp.float32)
                for k in range(D // 16):
                    u = ubuf[par, j, pl.ds(k * 16, 16)]
                    m = mbuf[par, j, pl.ds(k * 16, 16)]
                    acc = acc + u * m
                res = jnp.where(lane == e, jnp.sum(acc), res)
            out_v[pl.ds(c * CH + g * 16, 16)] = res

    def step(i, carry):
        for par in (0, 1):
            c = i * 2 + par

            @pl.when(c < NCH)
            def _():
                issue(c, par)

            @pl.when(jnp.logical_and(c > 0, c <= NCH))
            def _():
                drain(1 - par)
                compute(c - 1, 1 - par)

        return carry

    lax.fori_loop(0, (NCH + 2) // 2, step, 0)
    pltpu.sync_copy(out_v, out_hbm.at[pl.ds(base, BPW)])


@jax.jit
def _mf_kernel(user_ids, movie_ids, user_emb, movie_emb):
    uids = user_ids.astype(jnp.int32)
    mids = movie_ids.astype(jnp.int32)
    ut = user_emb.T
    mt = movie_emb.T

    mesh = plsc.VectorSubcoreMesh(core_axis_name="c", subcore_axis_name="s")

    gather = functools.partial(
        pl.kernel,
        mesh=mesh,
        compiler_params=pltpu.CompilerParams(needs_layout_passes=False),
        out_type=(jax.ShapeDtypeStruct((OROWS, 128), jnp.float32),
                  jax.ShapeDtypeStruct((OROWS, 128), jnp.float32)),
        scratch_types=[
            pltpu.VMEM((B,), jnp.int32),
            pltpu.VMEM((MCAP,), jnp.int32),
            pltpu.VMEM((MCAP,), jnp.int32),
            pltpu.VMEM((RING, D, 128), jnp.float32),
            pltpu.VMEM((D, UTAIL), jnp.float32),
            pltpu.VMEM((D, MTAIL), jnp.float32),
            pltpu.VMEM((16, 128), jnp.float32),
            pltpu.VMEM((16,), jnp.int32),
            pltpu.VMEM((16,), jnp.int32),
            pltpu.SemaphoreType.DMA,
            pltpu.SemaphoreType.DMA,
            pltpu.SemaphoreType.DMA,
            pltpu.SemaphoreType.DMA,
            pltpu.SemaphoreType.DMA,
        ],
    )(_gather_body)

    dot = functools.partial(
        pl.kernel,
        mesh=mesh,
        compiler_params=pltpu.CompilerParams(needs_layout_passes=False),
        out_type=jax.ShapeDtypeStruct((B,), jnp.float32),
        scratch_types=[
            pltpu.VMEM((2, 128, 128), jnp.float32),
            pltpu.VMEM((2, 128, 128), jnp.float32),
            pltpu.VMEM((BPW,), jnp.float32),
            pltpu.SemaphoreType.DMA,
            pltpu.SemaphoreType.DMA,
        ],
    )(_dot_body)

    uvecs, mvecs = gather(ut, mt, uids, mids)
    return dot(uvecs, mvecs)


def kernel(user_ids, movie_ids, user_emb, movie_emb):
    return _mf_kernel(user_ids, movie_ids, user_emb, movie_emb)


# speedup vs baseline: 2.5238x; 1.6708x over previous
import functools

import jax
import jax.numpy as jnp
from jax import lax
from jax.experimental import pallas as pl
from jax.experimental.pallas import tpu as pltpu
from jax.experimental.pallas import tpu_sc as plsc

B = 16384
D = 64
NU = 1000000
NM = 100000
NC = 2
NS = 16
NW = NC * NS
BPW = B // NW

NBU = (NU + 127) // 128
NBM = (NM + 127) // 128
UTAIL = NU - (NBU - 1) * 128
MTAIL = NM - (NBM - 1) * 128
UBASE, UEXTRA = NBU // NW, NBU % NW
MBASE, MEXTRA = NBM // NW, NBM % NW
RING = 2
WBLK = 4
WCOLS = WBLK * 128
MCAP = 4096
DUMP = B
OROWS = B + 16


def _gather_body(ut, mt, uids_h, mids_h, uvecs, mvecs,
                 ids_all, match_ids, match_pos, chunk, tailu_v, tailm_v,
                 stage_v, sidx_v, out_v_unused,
                 semr0, semr1, semr2, semr3, sem_s):
    del out_v_unused
    w = lax.axis_index("c") * NS + lax.axis_index("s")
    lane = lax.iota(jnp.int32, 16)
    sems = (semr0, semr1, semr2, semr3)

    def sum16(vec):
        return jnp.sum(vec)

    def extract(vec, onehot):
        return jnp.sum(jnp.where(onehot, vec, 0))

    def phase(table, ids_src, out_ref, nb_all, base, extra, tail_w, tail_v):
        blk0 = w * base + jnp.minimum(w, extra)
        nblk = base + jnp.where(w < extra, 1, 0)
        owns_tail = blk0 + nblk >= nb_all
        nfull = nblk - jnp.where(owns_tail, 1, 0)

        pltpu.sync_copy(ids_src, ids_all)

        lo = blk0 * 128
        hi = (blk0 + nblk) * 128

        def mstep(i, offv):
            v = ids_all[pl.ds(i * 16, 16)]
            m = jnp.logical_and(v >= lo, v < hi)
            csum = plsc.cumsum(m.astype(jnp.int32))
            idx = offv + csum - 1
            m = jnp.logical_and(m, idx < MCAP)
            plsc.store_scatter(match_ids, [idx], v, mask=m)
            plsc.store_scatter(match_pos, [idx], i * 16 + lane, mask=m)
            return offv + plsc.all_reduce_population_count(m)

        offv = lax.fori_loop(0, B // 16, mstep, jnp.zeros((16,), jnp.int32))
        nm_s = sum16(jnp.where(lane == 0, offv, 0))
        nmv = offv

        def process(gat, c0, width, carry, active):
            sidx_vec, bcv = carry
            nv = jnp.where(active, (nm_s + 15) // 16, 0)

            def vstep(v, carry):
                sidx_vec, bcv = carry
                ids16 = match_ids[pl.ds(v * 16, 16)]
                pos16 = match_pos[pl.ds(v * 16, 16)]
                valid = (v * 16 + lane) < nmv
                inblk = jnp.logical_and(
                    valid,
                    jnp.logical_and(ids16 >= c0, ids16 < c0 + width))

                def wcond(st):
                    msk, _, _ = st
                    return jnp.sum(jnp.where(msk, 1, 0)) > 0

                def wbody(st):
                    msk, sidx_vec, bcv = st
                    f = plsc.all_reduce_ffs(msk)
                    oh = lane == f
                    id_s = extract(ids16, oh)
                    b_s = extract(pos16, oh)
                    lc = id_s - c0
                    bc_s = sum16(jnp.where(lane == 0, bcv, 0))
                    lcv = jnp.broadcast_to(lc, (16,))
                    for k4 in range(D // 16):
                        g = gat(k4 * 16 + lane, lcv)
                        stage_v[bc_s, pl.ds(k4 * 16, 16)] = g
                    sidx_vec = jnp.where(lane == bcv, b_s, sidx_vec)
                    bcv = bcv + 1

                    @pl.when(bc_s == 15)
                    def _():
                        sidx_v[pl.ds(0, 16)] = sidx_vec
                        pltpu.async_copy(stage_v, out_ref.at[sidx_v],
                                         sem_s).wait()

                    sidx_vec = jnp.where(bc_s == 15,
                                         jnp.full((16,), DUMP, jnp.int32),
                                         sidx_vec)
                    bcv = jnp.where(bc_s == 15, 0, bcv)
                    return (jnp.logical_and(msk, jnp.logical_not(oh)),
                            sidx_vec, bcv)

                _, sidx_vec, bcv = lax.while_loop(
                    wcond, wbody, (inblk, sidx_vec, bcv))
                return (sidx_vec, bcv)

            return lax.fori_loop(0, nv, vstep, (sidx_vec, bcv))

        nwin = (nfull + WBLK - 1) // WBLK
        safe_last = nb_all - 1 - WBLK

        def wstart(j):
            return pl.multiple_of(
                jnp.minimum(blk0 + j * WBLK, safe_last) * 128, 128)

        def issue(slot, j):
            pltpu.async_copy(table.at[:, pl.ds(wstart(j), WCOLS)],
                             chunk.at[slot], sems[slot])

        for slot in range(RING):
            @pl.when(slot < nwin)
            def _():
                issue(slot, slot)

        def rstep(i, carry):
            for slot in range(RING):
                j = i * RING + slot

                @pl.when(j < nwin)
                def _():
                    pltpu.make_async_copy(
                        table.at[:, pl.ds(0, WCOLS)], chunk.at[slot],
                        sems[slot]).wait()

                slot_v = jnp.full((16,), slot, jnp.int32)
                carry = process(
                    lambda rows, cols, sv=slot_v: plsc.load_gather(
                        chunk, [sv, rows, cols]),
                    jnp.minimum(blk0 + j * WBLK, safe_last) * 128,
                    WCOLS, carry, j < nwin)

                @pl.when(j + RING < nwin)
                def _():
                    issue(slot, j + RING)
            return carry

        carry = (jnp.full((16,), DUMP, jnp.int32), jnp.zeros((16,), jnp.int32))
        nsteps = (nwin + RING - 1) // RING
        carry = lax.fori_loop(0, nsteps, rstep, carry)

        tc0 = (nb_all - 1) * 128

        @pl.when(owns_tail)
        def _():
            pltpu.sync_copy(table.at[:, pl.ds(tc0, tail_w)], tail_v)

        carry = process(
            lambda rows, cols: plsc.load_gather(tail_v, [rows, cols]),
            (nb_all - 1) * 128, tail_w, carry, owns_tail)

        sidx_vec, bcv = carry
        bc_s = sum16(jnp.where(lane == 0, bcv, 0))

        @pl.when(bc_s > 0)
        def _():
            sidx_v[pl.ds(0, 16)] = sidx_vec
            pltpu.async_copy(stage_v, out_ref.at[sidx_v], sem_s).wait()

    phase(ut, uids_h, uvecs, NBU, UBASE, UEXTRA, UTAIL, tailu_v)
    phase(mt, mids_h, mvecs, NBM, MBASE, MEXTRA, MTAIL, tailm_v)


def _dot_body(uvecs, mvecs, out_hbm, ubuf, mbuf, out_v, sem0, sem1):
    w = lax.axis_index("c") * NS + lax.axis_index("s")
    base = w * BPW
    CH = 128
    NCH = BPW // CH
    lane = lax.iota(jnp.int32, 16)
    sems = (sem0, sem1)

    def issue(c, par):
        pltpu.async_copy(uvecs.at[pl.ds(base + c * CH, CH)], ubuf.at[par],
                         sems[par])
        pltpu.async_copy(mvecs.at[pl.ds(base + c * CH, CH)], mbuf.at[par],
                         sems[par])

    def drain(par):
        pltpu.make_async_copy(uvecs.at[pl.ds(0, CH)], ubuf.at[par],
                              sems[par]).wait()
        pltpu.make_async_copy(mvecs.at[pl.ds(0, CH)], mbuf.at[par],
                              sems[par]).wait()

    def compute(c, par):
        for g in range(CH // 16):
            res = jnp.zeros((16,), jnp.float32)
            for e in range(16):
                j = g * 16 + e
                acc = jnp.zeros((16,), jnp.float32)
                for k in range(D // 16):
                    u = ubuf[par, j, pl.ds(k * 16, 16)]
                    m = mbuf[par, j, pl.ds(k * 16, 16)]
                    acc = acc + u * m
                res = jnp.where(lane == e, jnp.sum(acc), res)
            out_v[pl.ds(c * CH + g * 16, 16)] = res

    def step(i, carry):
        for par in (0, 1):
            c = i * 2 + par

            @pl.when(c < NCH)
            def _():
                issue(c, par)

            @pl.when(jnp.logical_and(c > 0, c <= NCH))
            def _():
                drain(1 - par)
                compute(c - 1, 1 - par)

        return carry

    lax.fori_loop(0, (NCH + 2) // 2, step, 0)
    pltpu.sync_copy(out_v, out_hbm.at[pl.ds(base, BPW)])


@jax.jit
def _mf_kernel(user_ids, movie_ids, user_emb, movie_emb):
    uids = user_ids.astype(jnp.int32)
    mids = movie_ids.astype(jnp.int32)
    ut = user_emb.T
    mt = movie_emb.T

    mesh = plsc.VectorSubcoreMesh(core_axis_name="c", subcore_axis_name="s")

    gather = functools.partial(
        pl.kernel,
        mesh=mesh,
        compiler_params=pltpu.CompilerParams(needs_layout_passes=False),
        out_type=(jax.ShapeDtypeStruct((OROWS, 128), jnp.float32),
                  jax.ShapeDtypeStruct((OROWS, 128), jnp.float32)),
        scratch_types=[
            pltpu.VMEM((B,), jnp.int32),
            pltpu.VMEM((MCAP,), jnp.int32),
            pltpu.VMEM((MCAP,), jnp.int32),
            pltpu.VMEM((RING, D, WCOLS), jnp.float32),
            pltpu.VMEM((D, UTAIL), jnp.float32),
            pltpu.VMEM((D, MTAIL), jnp.float32),
            pltpu.VMEM((16, 128), jnp.float32),
            pltpu.VMEM((16,), jnp.int32),
            pltpu.VMEM((16,), jnp.int32),
            pltpu.SemaphoreType.DMA,
            pltpu.SemaphoreType.DMA,
            pltpu.SemaphoreType.DMA,
            pltpu.SemaphoreType.DMA,
            pltpu.SemaphoreType.DMA,
        ],
    )(_gather_body)

    dot = functools.partial(
        pl.kernel,
        mesh=mesh,
        compiler_params=pltpu.CompilerParams(needs_layout_passes=False),
        out_type=jax.ShapeDtypeStruct((B,), jnp.float32),
        scratch_types=[
            pltpu.VMEM((2, 128, 128), jnp.float32),
            pltpu.VMEM((2, 128, 128), jnp.float32),
            pltpu.VMEM((BPW,), jnp.float32),
            pltpu.SemaphoreType.DMA,
            pltpu.SemaphoreType.DMA,
        ],
    )(_dot_body)

    uvecs, mvecs = gather(ut, mt, uids, mids)
    return dot(uvecs, mvecs)


def kernel(user_ids, movie_ids, user_emb, movie_emb):
    return _mf_kernel(user_ids, movie_ids, user_emb, movie_emb)


# speedup vs baseline: 2.5324x; 1.0034x over previous
import functools

import jax
import jax.numpy as jnp
from jax import lax
from jax.experimental import pallas as pl
from jax.experimental.pallas import tpu as pltpu
from jax.experimental.pallas import tpu_sc as plsc

B = 16384
D = 64
NU = 1000000
NM = 100000
NC = 2
NS = 16
NW = NC * NS
BPW = B // NW

NBU = (NU + 127) // 128
NBM = (NM + 127) // 128
UTAIL = NU - (NBU - 1) * 128
MTAIL = NM - (NBM - 1) * 128
UBASE, UEXTRA = NBU // NW, NBU % NW
MBASE, MEXTRA = NBM // NW, NBM % NW
RING = 2
WBLK = 4
WCOLS = WBLK * 128
MCAP = 4096
DUMP = B
OROWS = B + 16


def _gather_body(ut, mt, uids_h, mids_h, uvecs, mvecs,
                 ids_all, match_ids, match_pos, match2_ids, match2_pos,
                 chunk, tailu_v, tailm_v, stage_v, sidx_v,
                 semr0, semr1, semr2, semr3, sem_s):
    w = lax.axis_index("c") * NS + lax.axis_index("s")
    lane = lax.iota(jnp.int32, 16)
    sems = (semr0, semr1, semr2, semr3)

    def sum16(vec):
        return jnp.sum(vec)

    def extract(vec, onehot):
        return jnp.sum(jnp.where(onehot, vec, 0))

    def phase(table, ids_src, out_ref, nb_all, base, extra, tail_w, tail_v,
              n_sup, wps):
        blk0 = w * base + jnp.minimum(w, extra)
        nblk = base + jnp.where(w < extra, 1, 0)
        owns_tail = blk0 + nblk >= nb_all
        nfull = nblk - jnp.where(owns_tail, 1, 0)

        pltpu.sync_copy(ids_src, ids_all)

        lo = blk0 * 128
        hi = (blk0 + nblk) * 128

        def mstep(i, offv):
            v = ids_all[pl.ds(i * 16, 16)]
            m = jnp.logical_and(v >= lo, v < hi)
            csum = plsc.cumsum(m.astype(jnp.int32))
            idx = offv + csum - 1
            m = jnp.logical_and(m, idx < MCAP)
            plsc.store_scatter(match_ids, [idx], v, mask=m)
            plsc.store_scatter(match_pos, [idx], i * 16 + lane, mask=m)
            return offv + plsc.all_reduce_population_count(m)

        offv = lax.fori_loop(0, B // 16, mstep, jnp.zeros((16,), jnp.int32))
        nm_s = sum16(jnp.where(lane == 0, offv, 0))
        nmv = offv

        supcap = MCAP // n_sup

        def bstep(i, st):
            offv, r = st
            v = match_ids[pl.ds(i * 16, 16)]
            p = match_pos[pl.ds(i * 16, 16)]
            valid = (i * 16 + lane) < nmv
            sup = (v // 128 - blk0) // (wps * WBLK)
            m = jnp.logical_and(valid, sup == r)
            csum = plsc.cumsum(m.astype(jnp.int32))
            idx = r * supcap + offv + csum - 1
            m = jnp.logical_and(m, offv + csum - 1 < supcap)
            plsc.store_scatter(match2_ids, [idx], v, mask=m)
            plsc.store_scatter(match2_pos, [idx], p, mask=m)
            return (offv + plsc.all_reduce_population_count(m), r)

        nm2v = jnp.zeros((16,), jnp.int32)
        nv_all = (nm_s + 15) // 16
        for r in range(n_sup):
            offv, _ = lax.fori_loop(
                0, nv_all, bstep,
                (jnp.zeros((16,), jnp.int32), jnp.full((16,), r, jnp.int32)))
            nm2v = jnp.where(lane == r, offv, nm2v)

        def process(gat, c0, width, carry, active,
                    mids_ref, mpos_ref, lbase, cnt_s, cntv):
            sidx_vec, bcv = carry
            nv = jnp.where(active, (cnt_s + 15) // 16, 0)

            def vstep(v, carry):
                sidx_vec, bcv = carry
                ids16 = mids_ref[pl.ds(lbase + v * 16, 16)]
                pos16 = mpos_ref[pl.ds(lbase + v * 16, 16)]
                valid = (v * 16 + lane) < cntv
                inblk = jnp.logical_and(
                    valid,
                    jnp.logical_and(ids16 >= c0, ids16 < c0 + width))

                def wcond(st):
                    msk, _, _ = st
                    return jnp.sum(jnp.where(msk, 1, 0)) > 0

                def wbody(st):
                    msk, sidx_vec, bcv = st
                    f = plsc.all_reduce_ffs(msk)
                    oh = lane == f
                    id_s = extract(ids16, oh)
                    b_s = extract(pos16, oh)
                    lc = id_s - c0
                    bc_s = sum16(jnp.where(lane == 0, bcv, 0))
                    lcv = jnp.broadcast_to(lc, (16,))
                    for k4 in range(D // 16):
                        g = gat(k4 * 16 + lane, lcv)
                        stage_v[bc_s, pl.ds(k4 * 16, 16)] = g
                    sidx_vec = jnp.where(lane == bcv, b_s, sidx_vec)
                    bcv = bcv + 1

                    @pl.when(bc_s == 15)
                    def _():
                        sidx_v[pl.ds(0, 16)] = sidx_vec
                        pltpu.async_copy(stage_v, out_ref.at[sidx_v],
                                         sem_s).wait()

                    sidx_vec = jnp.where(bc_s == 15,
                                         jnp.full((16,), DUMP, jnp.int32),
                                         sidx_vec)
                    bcv = jnp.where(bc_s == 15, 0, bcv)
                    return (jnp.logical_and(msk, jnp.logical_not(oh)),
                            sidx_vec, bcv)

                _, sidx_vec, bcv = lax.while_loop(
                    wcond, wbody, (inblk, sidx_vec, bcv))
                return (sidx_vec, bcv)

            return lax.fori_loop(0, nv, vstep, (sidx_vec, bcv))

        nwin = (nfull + WBLK - 1) // WBLK
        safe_last = nb_all - 1 - WBLK

        def wstart(j):
            return pl.multiple_of(
                jnp.minimum(blk0 + j * WBLK, safe_last) * 128, 128)

        def issue(slot, j):
            pltpu.async_copy(table.at[:, pl.ds(wstart(j), WCOLS)],
                             chunk.at[slot], sems[slot])

        for slot in range(RING):
            @pl.when(slot < nwin)
            def _():
                issue(slot, slot)

        def rstep(i, carry):
            for slot in range(RING):
                j = i * RING + slot

                @pl.when(j < nwin)
                def _():
                    pltpu.make_async_copy(
                        table.at[:, pl.ds(0, WCOLS)], chunk.at[slot],
                        sems[slot]).wait()

                slot_v = jnp.full((16,), slot, jnp.int32)
                r = j // wps
                cnt_s = jnp.sum(jnp.where(lane == r, nm2v, 0))
                carry = process(
                    lambda rows, cols, sv=slot_v: plsc.load_gather(
                        chunk, [sv, rows, cols]),
                    jnp.minimum(blk0 + j * WBLK, safe_last) * 128,
                    WCOLS, carry, j < nwin,
                    match2_ids, match2_pos, r * supcap, cnt_s,
                    jnp.broadcast_to(cnt_s, (16,)))

                @pl.when(j + RING < nwin)
                def _():
                    issue(slot, j + RING)
            return carry

        carry = (jnp.full((16,), DUMP, jnp.int32), jnp.zeros((16,), jnp.int32))
        nsteps = (nwin + RING - 1) // RING
        carry = lax.fori_loop(0, nsteps, rstep, carry)

        tc0 = (nb_all - 1) * 128

        @pl.when(owns_tail)
        def _():
            pltpu.sync_copy(table.at[:, pl.ds(tc0, tail_w)], tail_v)

        carry = process(
            lambda rows, cols: plsc.load_gather(tail_v, [rows, cols]),
            (nb_all - 1) * 128, tail_w, carry, owns_tail,
            match_ids, match_pos, 0, nm_s, nmv)

        sidx_vec, bcv = carry
        bc_s = sum16(jnp.where(lane == 0, bcv, 0))

        @pl.when(bc_s > 0)
        def _():
            sidx_v[pl.ds(0, 16)] = sidx_vec
            pltpu.async_copy(stage_v, out_ref.at[sidx_v], sem_s).wait()

    phase(ut, uids_h, uvecs, NBU, UBASE, UEXTRA, UTAIL, tailu_v, 8, 8)
    phase(mt, mids_h, mvecs, NBM, MBASE, MEXTRA, MTAIL, tailm_v, 1, 64)


def _dot_body(uvecs, mvecs, out_hbm, ubuf, mbuf, out_v, sem0, sem1):
    w = lax.axis_index("c") * NS + lax.axis_index("s")
    base = w * BPW
    CH = 128
    NCH = BPW // CH
    lane = lax.iota(jnp.int32, 16)
    sems = (sem0, sem1)

    def issue(c, par):
        pltpu.async_copy(uvecs.at[pl.ds(base + c * CH, CH)], ubuf.at[par],
                         sems[par])
        pltpu.async_copy(mvecs.at[pl.ds(base + c * CH, CH)], mbuf.at[par],
                         sems[par])

    def drain(par):
        pltpu.make_async_copy(uvecs.at[pl.ds(0, CH)], ubuf.at[par],
                              sems[par]).wait()
        pltpu.make_async_copy(mvecs.at[pl.ds(0, CH)], mbuf.at[par],
                              sems[par]).wait()

    def compute(c, par):
        for g in range(CH // 16):
            res = jnp.zeros((16,), jnp.float32)
            for e in range(16):
                j = g * 16 + e
                acc = jnp.zeros((16,), jnp.float32)
                for k in range(D // 16):
                    u = ubuf[par, j, pl.ds(k * 16, 16)]
                    m = mbuf[par, j, pl.ds(k * 16, 16)]
                    acc = acc + u * m
                res = jnp.where(lane == e, jnp.sum(acc), res)
            out_v[pl.ds(c * CH + g * 16, 16)] = res

    def step(i, carry):
        for par in (0, 1):
            c = i * 2 + par

            @pl.when(c < NCH)
            def _():
                issue(c, par)

            @pl.when(jnp.logical_and(c > 0, c <= NCH))
            def _():
                drain(1 - par)
                compute(c - 1, 1 - par)

        return carry

    lax.fori_loop(0, (NCH + 2) // 2, step, 0)
    pltpu.sync_copy(out_v, out_hbm.at[pl.ds(base, BPW)])


@jax.jit
def _mf_kernel(user_ids, movie_ids, user_emb, movie_emb):
    uids = user_ids.astype(jnp.int32)
    mids = movie_ids.astype(jnp.int32)
    ut = user_emb.T
    mt = movie_emb.T

    mesh = plsc.VectorSubcoreMesh(core_axis_name="c", subcore_axis_name="s")

    gather = functools.partial(
        pl.kernel,
        mesh=mesh,
        compiler_params=pltpu.CompilerParams(needs_layout_passes=False),
        out_type=(jax.ShapeDtypeStruct((OROWS, 128), jnp.float32),
                  jax.ShapeDtypeStruct((OROWS, 128), jnp.float32)),
        scratch_types=[
            pltpu.VMEM((B,), jnp.int32),
            pltpu.VMEM((MCAP,), jnp.int32),
            pltpu.VMEM((MCAP,), jnp.int32),
            pltpu.VMEM((MCAP,), jnp.int32),
            pltpu.VMEM((MCAP,), jnp.int32),
            pltpu.VMEM((RING, D, WCOLS), jnp.float32),
            pltpu.VMEM((D, UTAIL), jnp.float32),
            pltpu.VMEM((D, MTAIL), jnp.float32),
            pltpu.VMEM((16, 128), jnp.float32),
            pltpu.VMEM((16,), jnp.int32),
            pltpu.SemaphoreType.DMA,
            pltpu.SemaphoreType.DMA,
            pltpu.SemaphoreType.DMA,
            pltpu.SemaphoreType.DMA,
            pltpu.SemaphoreType.DMA,
        ],
    )(_gather_body)

    dot = functools.partial(
        pl.kernel,
        mesh=mesh,
        compiler_params=pltpu.CompilerParams(needs_layout_passes=False),
        out_type=jax.ShapeDtypeStruct((B,), jnp.float32),
        scratch_types=[
            pltpu.VMEM((2, 128, 128), jnp.float32),
            pltpu.VMEM((2, 128, 128), jnp.float32),
            pltpu.VMEM((BPW,), jnp.float32),
            pltpu.SemaphoreType.DMA,
            pltpu.SemaphoreType.DMA,
        ],
    )(_dot_body)

    uvecs, mvecs = gather(ut, mt, uids, mids)
    return dot(uvecs, mvecs)


def kernel(user_ids, movie_ids, user_emb, movie_emb):
    return _mf_kernel(user_ids, movie_ids, user_emb, movie_emb)


# speedup vs baseline: 2.6015x; 1.0273x over previous
import functools

import jax
import jax.numpy as jnp
from jax import lax
from jax.experimental import pallas as pl
from jax.experimental.pallas import tpu as pltpu
from jax.experimental.pallas import tpu_sc as plsc

B = 16384
D = 64
NU = 1000000
NM = 100000
NC = 2
NS = 16
NW = NC * NS
BPW = B // NW

NBU = (NU + 127) // 128
NBM = (NM + 127) // 128
UTAIL = NU - (NBU - 1) * 128
MTAIL = NM - (NBM - 1) * 128
UBASE, UEXTRA = NBU // NW, NBU % NW
MBASE, MEXTRA = NBM // NW, NBM % NW
RING = 3
WBLK = 3
WCOLS = WBLK * 128
MCAP = 2048
DUMP = B
OROWS = B + 16


def _gather_body(ut, mt, uids_h, mids_h, uvecs, mvecs,
                 ids_all, match_ids, match_pos, match2_ids, match2_pos,
                 chunk, tailu_v, tailm_v, stage_v, sidx_v,
                 semr0, semr1, semr2, semr3, sem_s):
    w = lax.axis_index("c") * NS + lax.axis_index("s")
    lane = lax.iota(jnp.int32, 16)
    sems = (semr0, semr1, semr2, semr3)

    def sum16(vec):
        return jnp.sum(vec)

    def extract(vec, onehot):
        return jnp.sum(jnp.where(onehot, vec, 0))

    def phase(table, ids_src, out_ref, nb_all, base, extra, tail_w, tail_v,
              n_sup, wps):
        blk0 = w * base + jnp.minimum(w, extra)
        nblk = base + jnp.where(w < extra, 1, 0)
        owns_tail = blk0 + nblk >= nb_all
        nfull = nblk - jnp.where(owns_tail, 1, 0)

        pltpu.sync_copy(ids_src, ids_all)

        lo = blk0 * 128
        hi = (blk0 + nblk) * 128

        def mstep(i, offv):
            v = ids_all[pl.ds(i * 16, 16)]
            m = jnp.logical_and(v >= lo, v < hi)
            csum = plsc.cumsum(m.astype(jnp.int32))
            idx = offv + csum - 1
            m = jnp.logical_and(m, idx < MCAP)
            plsc.store_scatter(match_ids, [idx], v, mask=m)
            plsc.store_scatter(match_pos, [idx], i * 16 + lane, mask=m)
            return offv + plsc.all_reduce_population_count(m)

        offv = lax.fori_loop(0, B // 16, mstep, jnp.zeros((16,), jnp.int32))
        nm_s = sum16(jnp.where(lane == 0, offv, 0))
        nmv = offv

        supcap = MCAP // n_sup

        def bstep(i, st):
            offv, r = st
            v = match_ids[pl.ds(i * 16, 16)]
            p = match_pos[pl.ds(i * 16, 16)]
            valid = (i * 16 + lane) < nmv
            sup = (v // 128 - blk0) // (wps * WBLK)
            m = jnp.logical_and(valid, sup == r)
            csum = plsc.cumsum(m.astype(jnp.int32))
            idx = r * supcap + offv + csum - 1
            m = jnp.logical_and(m, offv + csum - 1 < supcap)
            plsc.store_scatter(match2_ids, [idx], v, mask=m)
            plsc.store_scatter(match2_pos, [idx], p, mask=m)
            return (offv + plsc.all_reduce_population_count(m), r)

        nm2v = jnp.zeros((16,), jnp.int32)
        nv_all = (nm_s + 15) // 16
        for r in range(n_sup):
            offv, _ = lax.fori_loop(
                0, nv_all, bstep,
                (jnp.zeros((16,), jnp.int32), jnp.full((16,), r, jnp.int32)))
            nm2v = jnp.where(lane == r, offv, nm2v)

        def process(gat, c0, width, carry, active,
                    mids_ref, mpos_ref, lbase, cnt_s, cntv):
            sidx_vec, bcv = carry
            nv = jnp.where(active, (cnt_s + 15) // 16, 0)

            def vstep(v, carry):
                sidx_vec, bcv = carry
                ids16 = mids_ref[pl.ds(lbase + v * 16, 16)]
                pos16 = mpos_ref[pl.ds(lbase + v * 16, 16)]
                valid = (v * 16 + lane) < cntv
                inblk = jnp.logical_and(
                    valid,
                    jnp.logical_and(ids16 >= c0, ids16 < c0 + width))

                def wcond(st):
                    msk, _, _ = st
                    return jnp.sum(jnp.where(msk, 1, 0)) > 0

                def wbody(st):
                    msk, sidx_vec, bcv = st
                    f = plsc.all_reduce_ffs(msk)
                    oh = lane == f
                    id_s = extract(ids16, oh)
                    b_s = extract(pos16, oh)
                    lc = id_s - c0
                    bc_s = sum16(jnp.where(lane == 0, bcv, 0))
                    lcv = jnp.broadcast_to(lc, (16,))
                    for k4 in range(D // 16):
                        g = gat(k4 * 16 + lane, lcv)
                        stage_v[bc_s, pl.ds(k4 * 16, 16)] = g
                    sidx_vec = jnp.where(lane == bcv, b_s, sidx_vec)
                    bcv = bcv + 1

                    @pl.when(bc_s == 15)
                    def _():
                        sidx_v[pl.ds(0, 16)] = sidx_vec
                        pltpu.async_copy(stage_v, out_ref.at[sidx_v],
                                         sem_s).wait()

                    sidx_vec = jnp.where(bc_s == 15,
                                         jnp.full((16,), DUMP, jnp.int32),
                                         sidx_vec)
                    bcv = jnp.where(bc_s == 15, 0, bcv)
                    return (jnp.logical_and(msk, jnp.logical_not(oh)),
                            sidx_vec, bcv)

                _, sidx_vec, bcv = lax.while_loop(
                    wcond, wbody, (inblk, sidx_vec, bcv))
                return (sidx_vec, bcv)

            return lax.fori_loop(0, nv, vstep, (sidx_vec, bcv))

        nwin = (nfull + WBLK - 1) // WBLK
        safe_last = nb_all - 1 - WBLK

        def wstart(j):
            return pl.multiple_of(
                jnp.minimum(blk0 + j * WBLK, safe_last) * 128, 128)

        def issue(slot, j):
            pltpu.async_copy(table.at[:, pl.ds(wstart(j), WCOLS)],
                             chunk.at[slot], sems[slot])

        for slot in range(RING):
            @pl.when(slot < nwin)
            def _():
                issue(slot, slot)

        def rstep(i, carry):
            for slot in range(RING):
                j = i * RING + slot

                @pl.when(j < nwin)
                def _():
                    pltpu.make_async_copy(
                        table.at[:, pl.ds(0, WCOLS)], chunk.at[slot],
                        sems[slot]).wait()

                slot_v = jnp.full((16,), slot, jnp.int32)
                r = j // wps
                cnt_s = jnp.sum(jnp.where(lane == r, nm2v, 0))
                carry = process(
                    lambda rows, cols, sv=slot_v: plsc.load_gather(
                        chunk, [sv, rows, cols]),
                    jnp.minimum(blk0 + j * WBLK, safe_last) * 128,
                    WCOLS, carry, j < nwin,
                    match2_ids, match2_pos, r * supcap, cnt_s,
                    jnp.broadcast_to(cnt_s, (16,)))

                @pl.when(j + RING < nwin)
                def _():
                    issue(slot, j + RING)
            return carry

        carry = (jnp.full((16,), DUMP, jnp.int32), jnp.zeros((16,), jnp.int32))
        nsteps = (nwin + RING - 1) // RING
        carry = lax.fori_loop(0, nsteps, rstep, carry)

        tc0 = (nb_all - 1) * 128

        @pl.when(owns_tail)
        def _():
            pltpu.sync_copy(table.at[:, pl.ds(tc0, tail_w)], tail_v)

        carry = process(
            lambda rows, cols: plsc.load_gather(tail_v, [rows, cols]),
            (nb_all - 1) * 128, tail_w, carry, owns_tail,
            match_ids, match_pos, 0, nm_s, nmv)

        sidx_vec, bcv = carry
        bc_s = sum16(jnp.where(lane == 0, bcv, 0))

        @pl.when(bc_s > 0)
        def _():
            sidx_v[pl.ds(0, 16)] = sidx_vec
            pltpu.async_copy(stage_v, out_ref.at[sidx_v], sem_s).wait()

    phase(ut, uids_h, uvecs, NBU, UBASE, UEXTRA, UTAIL, tailu_v, 8, 11)
    phase(mt, mids_h, mvecs, NBM, MBASE, MEXTRA, MTAIL, tailm_v, 1, 64)


def _dot_body(uvecs, mvecs, out_hbm, ubuf, mbuf, out_v, sem0, sem1):
    w = lax.axis_index("c") * NS + lax.axis_index("s")
    base = w * BPW
    CH = 128
    NCH = BPW // CH
    lane = lax.iota(jnp.int32, 16)
    sems = (sem0, sem1)

    def issue(c, par):
        pltpu.async_copy(uvecs.at[pl.ds(base + c * CH, CH)], ubuf.at[par],
                         sems[par])
        pltpu.async_copy(mvecs.at[pl.ds(base + c * CH, CH)], mbuf.at[par],
                         sems[par])

    def drain(par):
        pltpu.make_async_copy(uvecs.at[pl.ds(0, CH)], ubuf.at[par],
                              sems[par]).wait()
        pltpu.make_async_copy(mvecs.at[pl.ds(0, CH)], mbuf.at[par],
                              sems[par]).wait()

    def compute(c, par):
        for g in range(CH // 16):
            res = jnp.zeros((16,), jnp.float32)
            for e in range(16):
                j = g * 16 + e
                acc = jnp.zeros((16,), jnp.float32)
                for k in range(D // 16):
                    u = ubuf[par, j, pl.ds(k * 16, 16)]
                    m = mbuf[par, j, pl.ds(k * 16, 16)]
                    acc = acc + u * m
                res = jnp.where(lane == e, jnp.sum(acc), res)
            out_v[pl.ds(c * CH + g * 16, 16)] = res

    def step(i, carry):
        for par in (0, 1):
            c = i * 2 + par

            @pl.when(c < NCH)
            def _():
                issue(c, par)

            @pl.when(jnp.logical_and(c > 0, c <= NCH))
            def _():
                drain(1 - par)
                compute(c - 1, 1 - par)

        return carry

    lax.fori_loop(0, (NCH + 2) // 2, step, 0)
    pltpu.sync_copy(out_v, out_hbm.at[pl.ds(base, BPW)])


@jax.jit
def _mf_kernel(user_ids, movie_ids, user_emb, movie_emb):
    uids = user_ids.astype(jnp.int32)
    mids = movie_ids.astype(jnp.int32)
    ut = user_emb.T
    mt = movie_emb.T

    mesh = plsc.VectorSubcoreMesh(core_axis_name="c", subcore_axis_name="s")

    gather = functools.partial(
        pl.kernel,
        mesh=mesh,
        compiler_params=pltpu.CompilerParams(needs_layout_passes=False),
        out_type=(jax.ShapeDtypeStruct((OROWS, 128), jnp.float32),
                  jax.ShapeDtypeStruct((OROWS, 128), jnp.float32)),
        scratch_types=[
            pltpu.VMEM((B,), jnp.int32),
            pltpu.VMEM((MCAP,), jnp.int32),
            pltpu.VMEM((MCAP,), jnp.int32),
            pltpu.VMEM((MCAP,), jnp.int32),
            pltpu.VMEM((MCAP,), jnp.int32),
            pltpu.VMEM((RING, D, WCOLS), jnp.float32),
            pltpu.VMEM((D, UTAIL), jnp.float32),
            pltpu.VMEM((D, MTAIL), jnp.float32),
            pltpu.VMEM((16, 128), jnp.float32),
            pltpu.VMEM((16,), jnp.int32),
            pltpu.SemaphoreType.DMA,
            pltpu.SemaphoreType.DMA,
            pltpu.SemaphoreType.DMA,
            pltpu.SemaphoreType.DMA,
            pltpu.SemaphoreType.DMA,
        ],
    )(_gather_body)

    dot = functools.partial(
        pl.kernel,
        mesh=mesh,
        compiler_params=pltpu.CompilerParams(needs_layout_passes=False),
        out_type=jax.ShapeDtypeStruct((B,), jnp.float32),
        scratch_types=[
            pltpu.VMEM((2, 128, 128), jnp.float32),
            pltpu.VMEM((2, 128, 128), jnp.float32),
            pltpu.VMEM((BPW,), jnp.float32),
            pltpu.SemaphoreType.DMA,
            pltpu.SemaphoreType.DMA,
        ],
    )(_dot_body)

    uvecs, mvecs = gather(ut, mt, uids, mids)
    return dot(uvecs, mvecs)


def kernel(user_ids, movie_ids, user_emb, movie_emb):
    return _mf_kernel(user_ids, movie_ids, user_emb, movie_emb)
